# baseline (device time: 79736 ns/iter reference)
import jax
import jax.numpy as jnp
from jax import lax
from jax.experimental import pallas as pl
from jax.experimental.pallas import tpu as pltpu

N_CHUNKS = 16


def kernel(O, Wo):
    B, S, H_loc, D = O.shape
    K = H_loc * D
    N = Wo.shape[1]
    s_half = S // 2
    n_half = N // 2
    rows = s_half // N_CHUNKS

    X = O.reshape(B, S, K)

    def body(x_ref, wo_ref, out_ref, send_buf, recv_buf,
             x_send_sems, x_recv_sems, y_send_sems, y_recv_sems):
        my_x = lax.axis_index("x")
        my_y = lax.axis_index("y")
        x_peer = (1 - my_x, my_y)
        y_peer = (my_x, 1 - my_y)

        barrier_sem = pltpu.get_barrier_semaphore()
        for nbr in (x_peer, y_peer):
            pl.semaphore_signal(
                barrier_sem, inc=1,
                device_id=nbr, device_id_type=pl.DeviceIdType.MESH,
            )
        pl.semaphore_wait(barrier_sem, 2)

        own_rows = my_x * s_half
        peer_rows = (1 - my_x) * s_half
        my_col = my_y * n_half

        def x_rdma(c):
            return pltpu.make_async_remote_copy(
                src_ref=send_buf.at[:, pl.ds(c * rows, rows), :],
                dst_ref=recv_buf.at[:, pl.ds(c * rows, rows), :],
                send_sem=x_send_sems.at[c],
                recv_sem=x_recv_sems.at[c],
                device_id=x_peer,
                device_id_type=pl.DeviceIdType.MESH,
            )

        def y_rdma(c):
            return pltpu.make_async_remote_copy(
                src_ref=out_ref.at[:, pl.ds(c * rows, rows), pl.ds(my_col, n_half)],
                dst_ref=out_ref.at[:, pl.ds(c * rows, rows), pl.ds(my_col, n_half)],
                send_sem=y_send_sems.at[c],
                recv_sem=y_recv_sems.at[c],
                device_id=y_peer,
                device_id_type=pl.DeviceIdType.MESH,
            )

        for c in range(N_CHUNKS):
            r0 = c * rows
            for b in range(B):
                send_buf[b, pl.ds(r0, rows), :] = jnp.dot(
                    x_ref[b, pl.ds(peer_rows + r0, rows), :],
                    wo_ref[:, pl.ds(my_col, n_half)],
                    preferred_element_type=jnp.float32,
                )
            x_rdma(c).start()

        for b in range(B):
            out_ref[b, :, pl.ds(my_col, n_half)] = jnp.dot(
                x_ref[b, pl.ds(own_rows, s_half), :],
                wo_ref[:, pl.ds(my_col, n_half)],
                preferred_element_type=jnp.float32,
            )

        for c in range(N_CHUNKS):
            r0 = c * rows
            x_rdma(c).wait_recv()
            for b in range(B):
                out_ref[b, pl.ds(r0, rows), pl.ds(my_col, n_half)] = (
                    out_ref[b, pl.ds(r0, rows), pl.ds(my_col, n_half)]
                    + recv_buf[b, pl.ds(r0, rows), :]
                )
            y_rdma(c).start()

        for c in range(N_CHUNKS):
            y_rdma(c).wait_recv()
        for c in range(N_CHUNKS):
            x_rdma(c).wait_send()
            y_rdma(c).wait_send()

    return pl.pallas_call(
        body,
        out_shape=jax.ShapeDtypeStruct((B, s_half, N), jnp.float32),
        in_specs=[
            pl.BlockSpec(memory_space=pltpu.VMEM),
            pl.BlockSpec(memory_space=pltpu.VMEM),
        ],
        out_specs=pl.BlockSpec(memory_space=pltpu.VMEM),
        scratch_shapes=[
            pltpu.VMEM((B, s_half, n_half), jnp.float32),
            pltpu.VMEM((B, s_half, n_half), jnp.float32),
            pltpu.SemaphoreType.DMA((N_CHUNKS,)),
            pltpu.SemaphoreType.DMA((N_CHUNKS,)),
            pltpu.SemaphoreType.DMA((N_CHUNKS,)),
            pltpu.SemaphoreType.DMA((N_CHUNKS,)),
        ],
        compiler_params=pltpu.CompilerParams(collective_id=0),
    )(X, Wo)


# device time: 47604 ns/iter; 1.6750x vs baseline; 1.6750x over previous
import jax
import jax.numpy as jnp
from jax import lax
from jax.experimental import pallas as pl
from jax.experimental.pallas import tpu as pltpu

N_CHUNKS = 4


def kernel(O, Wo):
    B, S, H_loc, D = O.shape
    K = H_loc * D
    N = Wo.shape[1]
    s_half = S // 2
    n_half = N // 2
    rows = s_half // N_CHUNKS

    Xt = jnp.swapaxes(O.reshape(B, S, K), 1, 2)

    def body(xt_hbm, wo_hbm, out_hbm, xt_vmem, wo_f32, wo_bf, send_buf,
             recv_buf, y_send_buf, y_recv_buf, out_my, out_other,
             xt_sem, wo_sem,
             x_send_sems, x_recv_sems, y_send_sems, y_recv_sems,
             store_my_sems, store_other_sems):
        my_x = lax.axis_index("x")
        my_y = lax.axis_index("y")
        x_peer = (1 - my_x, my_y)
        y_peer = (my_x, 1 - my_y)

        own_rows = pl.multiple_of(my_x * s_half, 128)
        peer_rows = pl.multiple_of((1 - my_x) * s_half, 128)
        my_col = my_y * n_half
        other_col = (1 - my_y) * n_half

        def xt_copy():
            return pltpu.make_async_copy(xt_hbm, xt_vmem, xt_sem)

        def wo_copy():
            return pltpu.make_async_copy(
                wo_hbm.at[:, pl.ds(my_col, n_half)], wo_f32, wo_sem,
            )

        def bf(v):
            return v.astype(jnp.bfloat16)

        def kdot(lhs, rhs):
            return lax.dot_general(
                lhs, rhs, (((0,), (0,)), ((), ())),
                preferred_element_type=jnp.float32,
            )

        xt_copy().start()
        wo_copy().start()

        barrier_sem = pltpu.get_barrier_semaphore()
        for nbr in (x_peer, y_peer):
            pl.semaphore_signal(
                barrier_sem, inc=1,
                device_id=nbr, device_id_type=pl.DeviceIdType.MESH,
            )
        pl.semaphore_wait(barrier_sem, 2)

        wo_copy().wait()
        wo_bf[:, :] = bf(wo_f32[:, :])
        xt_copy().wait()

        def store_my(c):
            r0 = c * rows
            return pltpu.make_async_copy(
                out_my.at[:, pl.ds(r0, rows), :],
                out_hbm.at[:, pl.ds(r0, rows), pl.ds(my_col, n_half)],
                store_my_sems.at[c],
            )

        def store_other(c):
            r0 = c * rows
            return pltpu.make_async_copy(
                out_other.at[:, pl.ds(r0, rows), :],
                out_hbm.at[:, pl.ds(r0, rows), pl.ds(other_col, n_half)],
                store_other_sems.at[c],
            )

        def x_rdma(c):
            return pltpu.make_async_remote_copy(
                src_ref=send_buf.at[:, pl.ds(c * rows, rows), :],
                dst_ref=recv_buf.at[:, pl.ds(c * rows, rows), :],
                send_sem=x_send_sems.at[c],
                recv_sem=x_recv_sems.at[c],
                device_id=x_peer,
                device_id_type=pl.DeviceIdType.MESH,
            )

        def y_rdma(c):
            return pltpu.make_async_remote_copy(
                src_ref=y_send_buf.at[:, pl.ds(c * rows, rows), :],
                dst_ref=y_recv_buf.at[:, pl.ds(c * rows, rows), :],
                send_sem=y_send_sems.at[c],
                recv_sem=y_recv_sems.at[c],
                device_id=y_peer,
                device_id_type=pl.DeviceIdType.MESH,
            )

        for c in range(N_CHUNKS):
            r0 = c * rows
            for b in range(B):
                send_buf[b, pl.ds(r0, rows), :] = bf(kdot(
                    bf(xt_vmem[b, :, pl.ds(pl.multiple_of(peer_rows + r0, 128), rows)]),
                    wo_bf[:, :],
                ))
            x_rdma(c).start()

        for b in range(B):
            out_my[b, :, :] = kdot(
                bf(xt_vmem[b, :, pl.ds(pl.multiple_of(own_rows, 128), s_half)]),
                wo_bf[:, :],
            )

        for c in range(N_CHUNKS):
            r0 = c * rows
            x_rdma(c).wait_recv()
            for b in range(B):
                summed = (
                    out_my[b, pl.ds(r0, rows), :]
                    + recv_buf[b, pl.ds(r0, rows), :].astype(jnp.float32)
                )
                out_my[b, pl.ds(r0, rows), :] = summed
                y_send_buf[b, pl.ds(r0, rows), :] = bf(summed)
            y_rdma(c).start()
            store_my(c).start()

        for c in range(N_CHUNKS):
            r0 = c * rows
            y_rdma(c).wait_recv()
            for b in range(B):
                out_other[b, pl.ds(r0, rows), :] = (
                    y_recv_buf[b, pl.ds(r0, rows), :].astype(jnp.float32)
                )
            store_other(c).start()
        for c in range(N_CHUNKS):
            x_rdma(c).wait_send()
            y_rdma(c).wait_send()
            store_my(c).wait()
            store_other(c).wait()

    return pl.pallas_call(
        body,
        out_shape=jax.ShapeDtypeStruct((B, s_half, N), jnp.float32),
        in_specs=[
            pl.BlockSpec(memory_space=pltpu.MemorySpace.HBM),
            pl.BlockSpec(memory_space=pltpu.MemorySpace.HBM),
        ],
        out_specs=pl.BlockSpec(memory_space=pltpu.MemorySpace.HBM),
        scratch_shapes=[
            pltpu.VMEM((B, K, S), jnp.float32),
            pltpu.VMEM((K, n_half), jnp.float32),
            pltpu.VMEM((K, n_half), jnp.bfloat16),
            pltpu.VMEM((B, s_half, n_half), jnp.bfloat16),
            pltpu.VMEM((B, s_half, n_half), jnp.bfloat16),
            pltpu.VMEM((B, s_half, n_half), jnp.bfloat16),
            pltpu.VMEM((B, s_half, n_half), jnp.bfloat16),
            pltpu.VMEM((B, s_half, n_half), jnp.float32),
            pltpu.VMEM((B, s_half, n_half), jnp.float32),
            pltpu.SemaphoreType.DMA,
            pltpu.SemaphoreType.DMA,
            pltpu.SemaphoreType.DMA((N_CHUNKS,)),
            pltpu.SemaphoreType.DMA((N_CHUNKS,)),
            pltpu.SemaphoreType.DMA((N_CHUNKS,)),
            pltpu.SemaphoreType.DMA((N_CHUNKS,)),
            pltpu.SemaphoreType.DMA((N_CHUNKS,)),
            pltpu.SemaphoreType.DMA((N_CHUNKS,)),
        ],
        compiler_params=pltpu.CompilerParams(
            collective_id=0, vmem_limit_bytes=100 * 1024 * 1024,
        ),
    )(Xt, Wo)


# device time: 44267 ns/iter; 1.8013x vs baseline; 1.0754x over previous
import jax
import jax.numpy as jnp
from jax import lax
from jax.experimental import pallas as pl
from jax.experimental.pallas import tpu as pltpu

N_CHUNKS = 4
W_CHUNKS = 8


def kernel(O, Wo):
    B, S, H_loc, D = O.shape
    K = H_loc * D
    N = Wo.shape[1]
    s_half = S // 2
    n_half = N // 2
    rows = s_half // N_CHUNKS
    wrows = s_half // W_CHUNKS

    Xt = jnp.swapaxes(O.reshape(B, S, K), 1, 2)

    def body(xt_hbm, wo_hbm, out_hbm, xt_vmem, wo_f32, wo_bf, send_buf,
             recv_buf, y_send_buf, y_recv_buf, out_my, out_other,
             xt_sems, xt_own_sem, wo_sem,
             x_send_sems, x_recv_sems, y_send_sems, y_recv_sems,
             store_my_sems, store_other_sems):
        my_x = lax.axis_index("x")
        my_y = lax.axis_index("y")
        x_peer = (1 - my_x, my_y)
        y_peer = (my_x, 1 - my_y)

        own_rows = pl.multiple_of(my_x * s_half, 128)
        peer_rows = pl.multiple_of((1 - my_x) * s_half, 128)
        my_col = my_y * n_half
        other_col = (1 - my_y) * n_half

        def xt_peer_copy(c):
            r0 = peer_rows + c * rows
            return pltpu.make_async_copy(
                xt_hbm.at[:, :, pl.ds(r0, rows)],
                xt_vmem.at[:, :, pl.ds(r0, rows)],
                xt_sems.at[c],
            )

        def xt_own_copy():
            return pltpu.make_async_copy(
                xt_hbm.at[:, :, pl.ds(own_rows, s_half)],
                xt_vmem.at[:, :, pl.ds(own_rows, s_half)],
                xt_own_sem,
            )

        def wo_copy():
            return pltpu.make_async_copy(
                wo_hbm.at[:, pl.ds(my_col, n_half)], wo_f32, wo_sem,
            )

        def bf(v):
            return v.astype(jnp.bfloat16)

        def kdot(lhs, rhs):
            return lax.dot_general(
                lhs, rhs, (((0,), (0,)), ((), ())),
                preferred_element_type=jnp.float32,
            )

        for c in range(N_CHUNKS):
            xt_peer_copy(c).start()
        wo_copy().start()
        xt_own_copy().start()

        barrier_sem = pltpu.get_barrier_semaphore()
        for nbr in (x_peer, y_peer):
            pl.semaphore_signal(
                barrier_sem, inc=1,
                device_id=nbr, device_id_type=pl.DeviceIdType.MESH,
            )
        pl.semaphore_wait(barrier_sem, 2)

        wo_copy().wait()
        wo_bf[:, :] = bf(wo_f32[:, :])

        def store_my(c):
            r0 = c * wrows
            return pltpu.make_async_copy(
                out_my.at[:, pl.ds(r0, wrows), :],
                out_hbm.at[:, pl.ds(r0, wrows), pl.ds(my_col, n_half)],
                store_my_sems.at[c],
            )

        def store_other(c):
            r0 = c * wrows
            return pltpu.make_async_copy(
                out_other.at[:, pl.ds(r0, wrows), :],
                out_hbm.at[:, pl.ds(r0, wrows), pl.ds(other_col, n_half)],
                store_other_sems.at[c],
            )

        def x_rdma(c):
            return pltpu.make_async_remote_copy(
                src_ref=send_buf.at[:, pl.ds(c * wrows, wrows), :],
                dst_ref=recv_buf.at[:, pl.ds(c * wrows, wrows), :],
                send_sem=x_send_sems.at[c],
                recv_sem=x_recv_sems.at[c],
                device_id=x_peer,
                device_id_type=pl.DeviceIdType.MESH,
            )

        def y_rdma(c):
            return pltpu.make_async_remote_copy(
                src_ref=y_send_buf.at[:, pl.ds(c * wrows, wrows), :],
                dst_ref=y_recv_buf.at[:, pl.ds(c * wrows, wrows), :],
                send_sem=y_send_sems.at[c],
                recv_sem=y_recv_sems.at[c],
                device_id=y_peer,
                device_id_type=pl.DeviceIdType.MESH,
            )

        for c in range(N_CHUNKS):
            r0 = c * rows
            xt_peer_copy(c).wait()
            for b in range(B):
                send_buf[b, pl.ds(r0, rows), :] = bf(kdot(
                    bf(xt_vmem[b, :, pl.ds(pl.multiple_of(peer_rows + r0, 128), rows)]),
                    wo_bf[:, :],
                ))
            for w in range(W_CHUNKS // N_CHUNKS):
                x_rdma(c * (W_CHUNKS // N_CHUNKS) + w).start()

        xt_own_copy().wait()
        for b in range(B):
            out_my[b, :, :] = kdot(
                bf(xt_vmem[b, :, pl.ds(pl.multiple_of(own_rows, 128), s_half)]),
                wo_bf[:, :],
            )

        for c in range(W_CHUNKS):
            r0 = c * wrows
            x_rdma(c).wait_recv()
            for b in range(B):
                summed = (
                    out_my[b, pl.ds(r0, wrows), :]
                    + recv_buf[b, pl.ds(r0, wrows), :].astype(jnp.float32)
                )
                out_my[b, pl.ds(r0, wrows), :] = summed
                y_send_buf[b, pl.ds(r0, wrows), :] = bf(summed)
            y_rdma(c).start()
            store_my(c).start()

        for c in range(W_CHUNKS):
            r0 = c * wrows
            y_rdma(c).wait_recv()
            for b in range(B):
                out_other[b, pl.ds(r0, wrows), :] = (
                    y_recv_buf[b, pl.ds(r0, wrows), :].astype(jnp.float32)
                )
            store_other(c).start()
        for c in range(W_CHUNKS):
            x_rdma(c).wait_send()
            y_rdma(c).wait_send()
            store_my(c).wait()
            store_other(c).wait()

    return pl.pallas_call(
        body,
        out_shape=jax.ShapeDtypeStruct((B, s_half, N), jnp.float32),
        in_specs=[
            pl.BlockSpec(memory_space=pltpu.MemorySpace.HBM),
            pl.BlockSpec(memory_space=pltpu.MemorySpace.HBM),
        ],
        out_specs=pl.BlockSpec(memory_space=pl.ANY),
        scratch_shapes=[
            pltpu.VMEM((B, K, S), jnp.float32),
            pltpu.VMEM((K, n_half), jnp.float32),
            pltpu.VMEM((K, n_half), jnp.bfloat16),
            pltpu.VMEM((B, s_half, n_half), jnp.bfloat16),
            pltpu.VMEM((B, s_half, n_half), jnp.bfloat16),
            pltpu.VMEM((B, s_half, n_half), jnp.bfloat16),
            pltpu.VMEM((B, s_half, n_half), jnp.bfloat16),
            pltpu.VMEM((B, s_half, n_half), jnp.float32),
            pltpu.VMEM((B, s_half, n_half), jnp.float32),
            pltpu.SemaphoreType.DMA((N_CHUNKS,)),
            pltpu.SemaphoreType.DMA,
            pltpu.SemaphoreType.DMA,
            pltpu.SemaphoreType.DMA((W_CHUNKS,)),
            pltpu.SemaphoreType.DMA((W_CHUNKS,)),
            pltpu.SemaphoreType.DMA((W_CHUNKS,)),
            pltpu.SemaphoreType.DMA((W_CHUNKS,)),
            pltpu.SemaphoreType.DMA((W_CHUNKS,)),
            pltpu.SemaphoreType.DMA((W_CHUNKS,)),
        ],
        compiler_params=pltpu.CompilerParams(
            collective_id=0, vmem_limit_bytes=100 * 1024 * 1024,
        ),
    )(Xt, Wo)
